# Initial kernel scaffold; baseline (speedup 1.0000x reference)
import jax
import jax.numpy as jnp
from jax.experimental import pallas as pl


def kernel(encoded_patch_input, codebook_weight):
    raise NotImplementedError("write your pallas kernel here")


# speedup vs baseline: 1.4503x; 1.4503x over previous
import functools

import jax
import jax.numpy as jnp
from jax import lax
from jax.experimental import pallas as pl
from jax.experimental.pallas import tpu as pltpu
from jax.experimental.pallas import tpu_sc as plsc

NUM_CODE = 8192
CODE_DIM = 256
COMMITMENT_COST = 0.25
NUM_TOK = 8192

BM = 1024
BN = 1024

SC_CORES = 2
SC_SUBCORES = 16
SC_WORKERS = SC_CORES * SC_SUBCORES
TOK_PER_WORKER = NUM_TOK // SC_WORKERS


def _normalize_body(x_ref, y_ref, xb_ref, yo_ref, yb_ref):
    x = x_ref[...]
    n = jnp.sqrt(jnp.sum(x * x, axis=1, keepdims=True))
    xb_ref[...] = (x / jnp.maximum(n, 1e-12)).astype(jnp.bfloat16)
    y = y_ref[...]
    n = jnp.sqrt(jnp.sum(y * y, axis=1, keepdims=True))
    y_n = y / jnp.maximum(n, 1e-12)
    yo_ref[...] = y_n
    yb_ref[...] = y_n.astype(jnp.bfloat16)


def _normalize(enc, cb):
    grid = 8
    blk = pl.BlockSpec((NUM_TOK // grid, CODE_DIM), lambda i: (i, 0))
    return pl.pallas_call(
        _normalize_body,
        grid=(grid,),
        in_specs=[blk, blk],
        out_specs=[blk, blk, blk],
        out_shape=[
            jax.ShapeDtypeStruct((NUM_TOK, CODE_DIM), jnp.bfloat16),
            jax.ShapeDtypeStruct((NUM_CODE, CODE_DIM), jnp.float32),
            jax.ShapeDtypeStruct((NUM_CODE, CODE_DIM), jnp.bfloat16),
        ],
    )(enc, cb)


def _argmax_body(enc_ref, cb_ref, idx_ref, sim_ref, m_scr, i_scr):
    n = pl.program_id(1)
    s = lax.dot_general(
        enc_ref[...], cb_ref[...],
        (((1,), (1,)), ((), ())),
        preferred_element_type=jnp.float32,
    )
    local_max = jnp.max(s, axis=1, keepdims=True)
    lane = lax.broadcasted_iota(jnp.int32, s.shape, 1) + n * BN
    cand = jnp.where(s == local_max, lane, jnp.int32(2**30))
    local_idx = jnp.min(cand, axis=1, keepdims=True)

    @pl.when(n == 0)
    def _():
        m_scr[...] = local_max
        i_scr[...] = local_idx

    @pl.when(n > 0)
    def _():
        better = local_max > m_scr[...]
        i_scr[...] = jnp.where(better, local_idx, i_scr[...])
        m_scr[...] = jnp.maximum(local_max, m_scr[...])

    @pl.when(n == pl.num_programs(1) - 1)
    def _():
        idx_ref[...] = i_scr[...]
        sim_ref[...] = m_scr[...]


def _argmax_similarity(enc_n, cb_n):
    grid = (NUM_TOK // BM, NUM_CODE // BN)
    return pl.pallas_call(
        _argmax_body,
        grid=grid,
        in_specs=[
            pl.BlockSpec((BM, CODE_DIM), lambda t, n: (t, 0)),
            pl.BlockSpec((BN, CODE_DIM), lambda t, n: (n, 0)),
        ],
        compiler_params=pltpu.CompilerParams(
            dimension_semantics=("arbitrary", "arbitrary"),
        ),
        out_specs=[
            pl.BlockSpec((BM, 1), lambda t, n: (t, 0)),
            pl.BlockSpec((BM, 1), lambda t, n: (t, 0)),
        ],
        out_shape=[
            jax.ShapeDtypeStruct((NUM_TOK, 1), jnp.int32),
            jax.ShapeDtypeStruct((NUM_TOK, 1), jnp.float32),
        ],
        scratch_shapes=[
            pltpu.VMEM((BM, 1), jnp.float32),
            pltpu.VMEM((BM, 1), jnp.int32),
        ],
    )(enc_n, cb_n)


def _sc_gather_hist_body(cb_hbm, idx_hbm, quant_hbm, counts_hbm,
                         idx_v, rows_v, ones_v, stage_v, hist_sh, sem):
    c = lax.axis_index("c")
    s = lax.axis_index("s")
    wid = s * SC_CORES + c
    base = wid * TOK_PER_WORKER

    pltpu.sync_copy(idx_hbm.at[pl.ds(base, TOK_PER_WORKER)], idx_v)
    pltpu.async_copy(cb_hbm.at[idx_v], rows_v, sem).wait()
    pltpu.sync_copy(rows_v, quant_hbm.at[pl.ds(base, TOK_PER_WORKER)])

    for i in range(TOK_PER_WORKER // 16):
        ones_v[pl.ds(i * 16, 16)] = jnp.ones((16,), jnp.float32)

    @pl.when(s == 0)
    def _():
        def zero_chunk(i, _):
            stage_v[pl.ds(i * 16, 16)] = jnp.zeros((16,), jnp.float32)
            return 0
        lax.fori_loop(0, NUM_CODE // 16, zero_chunk, 0)
        pltpu.sync_copy(stage_v, hist_sh)

    plsc.subcore_barrier()
    pltpu.sync_copy(ones_v, hist_sh.at[idx_v], add=True)
    plsc.subcore_barrier()

    @pl.when(s == 0)
    def _():
        pltpu.sync_copy(hist_sh, counts_hbm.at[c])


def _sc_gather_hist(cb_n, idx):
    return pl.kernel(
        _sc_gather_hist_body,
        out_type=[
            jax.ShapeDtypeStruct((NUM_TOK, CODE_DIM), jnp.float32),
            jax.ShapeDtypeStruct((SC_CORES, NUM_CODE), jnp.float32),
        ],
        mesh=plsc.VectorSubcoreMesh(core_axis_name="c", subcore_axis_name="s"),
        scratch_types=[
            pltpu.VMEM((TOK_PER_WORKER,), jnp.int32),
            pltpu.VMEM((TOK_PER_WORKER, CODE_DIM), jnp.float32),
            pltpu.VMEM((TOK_PER_WORKER,), jnp.float32),
            pltpu.VMEM((NUM_CODE,), jnp.float32),
            pltpu.VMEM_SHARED((NUM_CODE,), jnp.float32),
            pltpu.SemaphoreType.DMA,
        ],
    )(cb_n, idx)


def _finalize_body(sim_ref, cnt_ref, loss_ref, perp_ref):
    s_sum = jnp.sum(sim_ref[...])
    denom = float(NUM_TOK) * float(CODE_DIM)
    loss = (1.0 + COMMITMENT_COST) * (2.0 * NUM_TOK - 2.0 * s_sum) / denom
    loss_ref[...] = loss.reshape(1, 1)

    cnt = cnt_ref[...]
    p = (cnt[0:1, :] + cnt[1:2, :]) * (1.0 / NUM_TOK)
    ent = -jnp.sum(p * jnp.log(p + 1e-10))
    perp_ref[...] = jnp.exp(ent).reshape(1, 1)


def _finalize(sim, counts):
    return pl.pallas_call(
        _finalize_body,
        out_shape=[
            jax.ShapeDtypeStruct((1, 1), jnp.float32),
            jax.ShapeDtypeStruct((1, 1), jnp.float32),
        ],
    )(sim, counts)


def kernel(encoded_patch_input, codebook_weight):
    enc = encoded_patch_input.reshape(NUM_TOK, CODE_DIM)
    enc_b, cb_n, cb_b = _normalize(enc, codebook_weight)
    idx, sim = _argmax_similarity(enc_b, cb_b)
    idx_flat = idx.reshape(NUM_TOK)
    quant, counts = _sc_gather_hist(cb_n, idx_flat)
    loss, perp = _finalize(sim.reshape(64, 128), counts)

    B, C, Tn = encoded_patch_input.shape[:3]
    return (
        loss.reshape(()),
        quant.reshape(B, C, Tn, CODE_DIM),
        perp.reshape(()),
        codebook_weight,
        idx_flat.reshape(B, C, Tn),
    )
